# baseline (speedup 1.0000x reference)
import functools

import jax
import jax.numpy as jnp
from jax import lax
from jax.experimental import pallas as pl
from jax.experimental.pallas import tpu as pltpu
from jax.experimental.pallas import tpu_sc as plsc

DM = 316
DPAD = 384
NG = 18
_DN0 = (((0,), (0,)), ((), ()))


def _proc_body(mt_ref, gt_ref, w1_ref, w2_ref, b_ref, ut_ref, out_ref, uout_ref):
    acc = lax.dot_general(mt_ref[...], w1_ref[...], _DN0,
                          preferred_element_type=jnp.float32)
    acc = acc + lax.dot_general(gt_ref[...], w2_ref[...], _DN0,
                                preferred_element_type=jnp.float32)
    out_ref[...] = jnp.maximum(acc + b_ref[...], 0.0)
    uout_ref[:, :DM] = ut_ref[...].T
    uout_ref[:, DM:] = jnp.zeros_like(uout_ref[:, DM:])


def _precompute(mt_t, gt_t, w1, w2, bpad, ut_t):
    v = mt_t.shape[1]
    rb = 1024
    return pl.pallas_call(
        _proc_body,
        grid=(pl.cdiv(v, rb),),
        in_specs=[
            pl.BlockSpec((DM, rb), lambda i: (0, i)),
            pl.BlockSpec((NG, rb), lambda i: (0, i)),
            pl.BlockSpec((DM, DPAD), lambda i: (0, 0)),
            pl.BlockSpec((NG, DPAD), lambda i: (0, 0)),
            pl.BlockSpec((1, DPAD), lambda i: (0, 0)),
            pl.BlockSpec((DM, rb), lambda i: (0, i)),
        ],
        out_specs=[
            pl.BlockSpec((rb, DPAD), lambda i: (i, 0)),
            pl.BlockSpec((rb, DPAD), lambda i: (i, 0)),
        ],
        out_shape=[
            jax.ShapeDtypeStruct((v, DPAD), jnp.float32),
            jax.ShapeDtypeStruct((v, DPAD), jnp.float32),
        ],
    )(mt_t, gt_t, w1, w2, bpad, ut_t)


def _sc_gather(p_tab, idx_flat, user_tab, user_id):
    nc, ns = 2, 16
    nw = nc * ns
    n = idx_flat.shape[0]
    per_w = n // nw
    ch = 112
    n_ch = per_w // ch
    b = user_id.shape[0]
    u_per_w = b // nw
    mesh = plsc.VectorSubcoreMesh(core_axis_name="c", subcore_axis_name="s",
                                  num_cores=nc, num_subcores=ns)

    @functools.partial(
        pl.kernel,
        mesh=mesh,
        out_type=[
            jax.ShapeDtypeStruct((n, DPAD), jnp.float32),
            jax.ShapeDtypeStruct((b, DPAD), jnp.float32),
        ],
        scratch_types=[
            pltpu.VMEM((ch,), jnp.int32),
            pltpu.VMEM((ch, DPAD), jnp.float32),
            pltpu.VMEM((u_per_w,), jnp.int32),
            pltpu.VMEM((u_per_w, DPAD), jnp.float32),
            pltpu.SemaphoreType.DMA,
        ],
    )
    def k(p_hbm, idx_hbm, utab_hbm, uid_hbm, g_hbm, u_hbm,
          idx_v, rows_v, uidx_v, urows_v, sem):
        wid = lax.axis_index("s") * nc + lax.axis_index("c")
        base = wid * per_w

        def body(c, carry):
            off = base + c * ch
            pltpu.sync_copy(idx_hbm.at[pl.ds(off, ch)], idx_v)
            pltpu.async_copy(p_hbm.at[idx_v], rows_v, sem).wait()
            pltpu.sync_copy(rows_v, g_hbm.at[pl.ds(off, ch)])
            return carry

        lax.fori_loop(0, n_ch, body, 0)

        ub = wid * u_per_w
        pltpu.sync_copy(uid_hbm.at[pl.ds(ub, u_per_w)], uidx_v)
        pltpu.async_copy(utab_hbm.at[uidx_v], urows_v, sem).wait()
        pltpu.sync_copy(urows_v, u_hbm.at[pl.ds(ub, u_per_w)])

    return k(p_tab, idx_flat, user_tab, user_id)


def _finish_body(g_ref, r_ref, p_ref, u_ref, sx_ref, ag_ref, oc_ref,
                 st_ref, at_ref, ot_ref, out1_ref, out2_ref):
    s = r_ref.shape[1]
    g = g_ref[...][:, :s, :DM]
    pos = p_ref[...]
    r = r_ref[...]
    out1_ref[...] = (g + pos[None, :, :]) * r[:, :, None]

    rb = u_ref.shape[0]

    def onehot_emb(x_ref, tab_ref, nv):
        x = x_ref[...]
        i = lax.broadcasted_iota(jnp.int32, (rb, nv), 1).astype(jnp.float32)
        oh = (x == i).astype(jnp.float32)
        return jnp.dot(oh, tab_ref[...], preferred_element_type=jnp.float32)

    e_s = onehot_emb(sx_ref, st_ref, 2)
    e_a = onehot_emb(ag_ref, at_ref, 7)
    e_o = onehot_emb(oc_ref, ot_ref, 21)
    out2_ref[...] = jnp.concatenate([u_ref[...][:, :DM], e_s, e_a, e_o], axis=1)


def _finish(g3, ratings_ext, pos_ext, u, sex_f, age_f, occ_f,
            sex_table, age_table, occ_table):
    b, sp, _ = g3.shape
    s = ratings_ext.shape[1]
    rb = 32
    return pl.pallas_call(
        _finish_body,
        grid=(b // rb,),
        in_specs=[
            pl.BlockSpec((rb, sp, DPAD), lambda i: (i, 0, 0)),
            pl.BlockSpec((rb, s), lambda i: (i, 0)),
            pl.BlockSpec((s, DM), lambda i: (0, 0)),
            pl.BlockSpec((rb, DPAD), lambda i: (i, 0)),
            pl.BlockSpec((rb, 1), lambda i: (i, 0)),
            pl.BlockSpec((rb, 1), lambda i: (i, 0)),
            pl.BlockSpec((rb, 1), lambda i: (i, 0)),
            pl.BlockSpec((2, 1), lambda i: (0, 0)),
            pl.BlockSpec((7, 2), lambda i: (0, 0)),
            pl.BlockSpec((21, 4), lambda i: (0, 0)),
        ],
        out_specs=[
            pl.BlockSpec((rb, s, DM), lambda i: (i, 0, 0)),
            pl.BlockSpec((rb, DM + 7), lambda i: (i, 0)),
        ],
        out_shape=[
            jax.ShapeDtypeStruct((b, s, DM), jnp.float32),
            jax.ShapeDtypeStruct((b, DM + 7), jnp.float32),
        ],
    )(g3, ratings_ext, pos_ext, u, sex_f, age_f, occ_f,
      sex_table, age_table, occ_table)


def kernel(user_id, sex, age_group, occupation, target_movie_id, sequence_movie_ids,
           sequence_ratings, user_id_table, sex_table, age_group_table, occupation_table,
           movie_table, genre_table, proc_W, proc_b, pos_table):
    b = user_id.shape[0]
    seq = pos_table.shape[0]

    w1 = jnp.pad(proc_W[:DM], ((0, 0), (0, DPAD - DM)))
    w2 = jnp.pad(proc_W[DM:], ((0, 0), (0, DPAD - DM)))
    bpad = jnp.pad(proc_b, (0, DPAD - DM)).reshape(1, DPAD)

    p_tab, u_tab = _precompute(movie_table.T, genre_table.T, w1, w2, bpad,
                               user_id_table.T)

    sp = ((seq + 7) // 8) * 8
    idx_all = jnp.concatenate(
        [sequence_movie_ids, target_movie_id, sequence_movie_ids[:, :sp - seq]],
        axis=1).astype(jnp.int32).reshape(b * sp)

    ratings_ext = jnp.concatenate(
        [sequence_ratings.astype(jnp.float32), jnp.ones((b, 1), jnp.float32)], axis=1)
    pos_ext = jnp.concatenate(
        [pos_table[:seq - 1], jnp.zeros((1, DM), jnp.float32)], axis=0)
    uid = user_id.astype(jnp.int32)
    sex_f = sex.astype(jnp.float32).reshape(b, 1)
    age_f = age_group.astype(jnp.float32).reshape(b, 1)
    occ_f = occupation.astype(jnp.float32).reshape(b, 1)

    nh = b // 2
    outs = []
    for h in range(2):
        idx_h = idx_all[h * nh * sp:(h + 1) * nh * sp]
        g2, u = _sc_gather(p_tab, idx_h, u_tab, uid[h * nh:(h + 1) * nh])
        o1, o2 = _finish(
            g2.reshape(nh, sp, DPAD),
            ratings_ext[h * nh:(h + 1) * nh], pos_ext, u,
            sex_f[h * nh:(h + 1) * nh], age_f[h * nh:(h + 1) * nh],
            occ_f[h * nh:(h + 1) * nh],
            sex_table, age_group_table, occupation_table)
        outs.append((o1, o2))
    out1 = jnp.concatenate([outs[0][0], outs[1][0]], axis=0)
    out2 = jnp.concatenate([outs[0][1], outs[1][1]], axis=0)
    return (out1, out2)


# speedup vs baseline: 1.2730x; 1.2730x over previous
import functools

import jax
import jax.numpy as jnp
from jax import lax
from jax.experimental import pallas as pl
from jax.experimental.pallas import tpu as pltpu
from jax.experimental.pallas import tpu_sc as plsc

DM = 316
DPAD = 384
NG = 18
_DN0 = (((0,), (0,)), ((), ()))


def _proc_body(mt_ref, gt_ref, w1_ref, w2_ref, b_ref, ut_ref, out_ref, uout_ref):
    acc = lax.dot_general(mt_ref[...], w1_ref[...], _DN0,
                          preferred_element_type=jnp.float32)
    acc = acc + lax.dot_general(gt_ref[...], w2_ref[...], _DN0,
                                preferred_element_type=jnp.float32)
    out_ref[...] = jnp.maximum(acc + b_ref[...], 0.0)
    uout_ref[:, :DM] = ut_ref[...].T
    uout_ref[:, DM:] = jnp.zeros_like(uout_ref[:, DM:])


def _precompute(mt_t, gt_t, w1, w2, bpad, ut_t):
    v = mt_t.shape[1]
    rb = 1024
    return pl.pallas_call(
        _proc_body,
        grid=(pl.cdiv(v, rb),),
        in_specs=[
            pl.BlockSpec((DM, rb), lambda i: (0, i)),
            pl.BlockSpec((NG, rb), lambda i: (0, i)),
            pl.BlockSpec((DM, DPAD), lambda i: (0, 0)),
            pl.BlockSpec((NG, DPAD), lambda i: (0, 0)),
            pl.BlockSpec((1, DPAD), lambda i: (0, 0)),
            pl.BlockSpec((DM, rb), lambda i: (0, i)),
        ],
        out_specs=[
            pl.BlockSpec((rb, DPAD), lambda i: (i, 0)),
            pl.BlockSpec((rb, DPAD), lambda i: (i, 0)),
        ],
        out_shape=[
            jax.ShapeDtypeStruct((v, DPAD), jnp.float32),
            jax.ShapeDtypeStruct((v, DPAD), jnp.float32),
        ],
    )(mt_t, gt_t, w1, w2, bpad, ut_t)


def _sc_gather(p_tab, idx_flat, user_tab, user_id):
    nc, ns = 2, 16
    nw = nc * ns
    n = idx_flat.shape[0]
    per_w = n // nw
    ch = 112
    n_ch = per_w // ch
    b = user_id.shape[0]
    u_per_w = b // nw
    mesh = plsc.VectorSubcoreMesh(core_axis_name="c", subcore_axis_name="s",
                                  num_cores=nc, num_subcores=ns)

    @functools.partial(
        pl.kernel,
        mesh=mesh,
        out_type=[
            jax.ShapeDtypeStruct((n, DPAD), jnp.float32),
            jax.ShapeDtypeStruct((b, DPAD), jnp.float32),
        ],
        scratch_types=[
            pltpu.VMEM((ch,), jnp.int32),
            pltpu.VMEM((ch, DPAD), jnp.float32),
            pltpu.VMEM((u_per_w,), jnp.int32),
            pltpu.VMEM((u_per_w, DPAD), jnp.float32),
            pltpu.SemaphoreType.DMA,
        ],
    )
    def k(p_hbm, idx_hbm, utab_hbm, uid_hbm, g_hbm, u_hbm,
          idx_v, rows_v, uidx_v, urows_v, sem):
        wid = lax.axis_index("s") * nc + lax.axis_index("c")
        base = wid * per_w

        def body(c, carry):
            off = base + c * ch
            pltpu.sync_copy(idx_hbm.at[pl.ds(off, ch)], idx_v)
            pltpu.async_copy(p_hbm.at[idx_v], rows_v, sem).wait()
            pltpu.sync_copy(rows_v, g_hbm.at[pl.ds(off, ch)])
            return carry

        lax.fori_loop(0, n_ch, body, 0)

        ub = wid * u_per_w
        pltpu.sync_copy(uid_hbm.at[pl.ds(ub, u_per_w)], uidx_v)
        pltpu.async_copy(utab_hbm.at[uidx_v], urows_v, sem).wait()
        pltpu.sync_copy(urows_v, u_hbm.at[pl.ds(ub, u_per_w)])

    return k(p_tab, idx_flat, user_tab, user_id)


_SG = 8
_BB = 128


def _finish1_body(g_ref, rt_ref, pt_ref, out_ref):
    ptt = pt_ref[...].T
    for si in range(_SG):
        gt = g_ref[:, si, :DM].T
        rv = rt_ref[si:si + 1, :]
        out_ref[si] = (gt + ptt[:, si:si + 1]) * rv


def _finish1(g3, ratings_t, pos_p):
    b, sp, _ = g3.shape
    s = ratings_t.shape[0]
    return pl.pallas_call(
        _finish1_body,
        grid=(sp // _SG, b // _BB),
        in_specs=[
            pl.BlockSpec((_BB, _SG, DPAD), lambda i, j: (j, i, 0)),
            pl.BlockSpec((_SG, _BB), lambda i, j: (i, j)),
            pl.BlockSpec((_SG, DM), lambda i, j: (i, 0)),
        ],
        out_specs=pl.BlockSpec((_SG, DM, _BB), lambda i, j: (i, 0, j)),
        out_shape=jax.ShapeDtypeStruct((s, DM, b), jnp.float32),
    )(g3, ratings_t, pos_p)


def _finish2_body(u_ref, sx_ref, ag_ref, oc_ref, st_ref, at_ref, ot_ref,
                  out2_ref):
    rb = u_ref.shape[0]

    def onehot_emb(x_ref, tab_ref, nv):
        x = x_ref[...]
        i = lax.broadcasted_iota(jnp.int32, (rb, nv), 1).astype(jnp.float32)
        oh = (x == i).astype(jnp.float32)
        return jnp.dot(oh, tab_ref[...], preferred_element_type=jnp.float32)

    e_s = onehot_emb(sx_ref, st_ref, 2)
    e_a = onehot_emb(ag_ref, at_ref, 7)
    e_o = onehot_emb(oc_ref, ot_ref, 21)
    out2_ref[...] = jnp.concatenate([u_ref[...][:, :DM], e_s, e_a, e_o], axis=1)


def _finish2(u, sex_f, age_f, occ_f, sex_table, age_table, occ_table):
    b = u.shape[0]
    rb = 128
    return pl.pallas_call(
        _finish2_body,
        grid=(b // rb,),
        in_specs=[
            pl.BlockSpec((rb, DPAD), lambda i: (i, 0)),
            pl.BlockSpec((rb, 1), lambda i: (i, 0)),
            pl.BlockSpec((rb, 1), lambda i: (i, 0)),
            pl.BlockSpec((rb, 1), lambda i: (i, 0)),
            pl.BlockSpec((2, 1), lambda i: (0, 0)),
            pl.BlockSpec((7, 2), lambda i: (0, 0)),
            pl.BlockSpec((21, 4), lambda i: (0, 0)),
        ],
        out_specs=pl.BlockSpec((rb, DM + 7), lambda i: (i, 0)),
        out_shape=jax.ShapeDtypeStruct((b, DM + 7), jnp.float32),
    )(u, sex_f, age_f, occ_f, sex_table, age_table, occ_table)


def kernel(user_id, sex, age_group, occupation, target_movie_id, sequence_movie_ids,
           sequence_ratings, user_id_table, sex_table, age_group_table, occupation_table,
           movie_table, genre_table, proc_W, proc_b, pos_table):
    b = user_id.shape[0]
    seq = pos_table.shape[0]

    w1 = jnp.pad(proc_W[:DM], ((0, 0), (0, DPAD - DM)))
    w2 = jnp.pad(proc_W[DM:], ((0, 0), (0, DPAD - DM)))
    bpad = jnp.pad(proc_b, (0, DPAD - DM)).reshape(1, DPAD)

    p_tab, u_tab = _precompute(movie_table.T, genre_table.T, w1, w2, bpad,
                               user_id_table.T)

    sp = ((seq + 7) // 8) * 8
    idx_all = jnp.concatenate(
        [sequence_movie_ids, target_movie_id, sequence_movie_ids[:, :sp - seq]],
        axis=1).astype(jnp.int32).reshape(b * sp)

    ratings_ext = jnp.concatenate(
        [sequence_ratings.astype(jnp.float32), jnp.ones((b, 1), jnp.float32)], axis=1)
    pos_ext = jnp.concatenate(
        [pos_table[:seq - 1], jnp.zeros((1, DM), jnp.float32)], axis=0)
    uid = user_id.astype(jnp.int32)
    sex_f = sex.astype(jnp.float32).reshape(b, 1)
    age_f = age_group.astype(jnp.float32).reshape(b, 1)
    occ_f = occupation.astype(jnp.float32).reshape(b, 1)

    ratings_t = ratings_ext.T
    pos_p = jnp.pad(pos_ext, ((0, sp - seq), (0, 0)))

    g2, u = _sc_gather(p_tab, idx_all, u_tab, uid)
    ot = _finish1(g2.reshape(b, sp, DPAD), ratings_t, pos_p)
    out1 = jnp.transpose(ot, (2, 0, 1))
    out2 = _finish2(u, sex_f, age_f, occ_f,
                    sex_table, age_group_table, occupation_table)
    return (out1, out2)


# speedup vs baseline: 1.3591x; 1.0676x over previous
import functools

import jax
import jax.numpy as jnp
from jax import lax
from jax.experimental import pallas as pl
from jax.experimental.pallas import tpu as pltpu
from jax.experimental.pallas import tpu_sc as plsc

DM = 316
DPAD = 384
NG = 18
_DN0 = (((0,), (0,)), ((), ()))


def _proc_body(mt_ref, gt_ref, w1_ref, w2_ref, b_ref, ut_ref, out_ref, uout_ref):
    acc = lax.dot_general(mt_ref[...], w1_ref[...], _DN0,
                          preferred_element_type=jnp.float32)
    acc = acc + lax.dot_general(gt_ref[...], w2_ref[...], _DN0,
                                preferred_element_type=jnp.float32)
    out_ref[...] = jnp.maximum(acc + b_ref[...], 0.0)
    uout_ref[:, :DM] = ut_ref[...].T
    uout_ref[:, DM:] = jnp.zeros_like(uout_ref[:, DM:])


def _precompute(mt_t, gt_t, w1, w2, bpad, ut_t):
    v = mt_t.shape[1]
    rb = 1024
    return pl.pallas_call(
        _proc_body,
        grid=(pl.cdiv(v, rb),),
        in_specs=[
            pl.BlockSpec((DM, rb), lambda i: (0, i)),
            pl.BlockSpec((NG, rb), lambda i: (0, i)),
            pl.BlockSpec((DM, DPAD), lambda i: (0, 0)),
            pl.BlockSpec((NG, DPAD), lambda i: (0, 0)),
            pl.BlockSpec((1, DPAD), lambda i: (0, 0)),
            pl.BlockSpec((DM, rb), lambda i: (0, i)),
        ],
        out_specs=[
            pl.BlockSpec((rb, DPAD), lambda i: (i, 0)),
            pl.BlockSpec((rb, DPAD), lambda i: (i, 0)),
        ],
        out_shape=[
            jax.ShapeDtypeStruct((v, DPAD), jnp.float32),
            jax.ShapeDtypeStruct((v, DPAD), jnp.float32),
        ],
    )(mt_t, gt_t, w1, w2, bpad, ut_t)


def _sc_gather(p_tab, idx_flat, user_tab, user_id):
    nc, ns = 2, 16
    nw = nc * ns
    n = idx_flat.shape[0]
    per_w = n // nw
    ch = 112
    n_ch = per_w // ch
    b = user_id.shape[0]
    u_per_w = b // nw
    mesh = plsc.VectorSubcoreMesh(core_axis_name="c", subcore_axis_name="s",
                                  num_cores=nc, num_subcores=ns)

    @functools.partial(
        pl.kernel,
        mesh=mesh,
        out_type=[
            jax.ShapeDtypeStruct((n, DPAD), jnp.float32),
            jax.ShapeDtypeStruct((b, DPAD), jnp.float32),
        ],
        scratch_types=[
            pltpu.VMEM((ch,), jnp.int32),
            pltpu.VMEM((ch,), jnp.int32),
            pltpu.VMEM((ch, DPAD), jnp.float32),
            pltpu.VMEM((ch, DPAD), jnp.float32),
            pltpu.VMEM((u_per_w,), jnp.int32),
            pltpu.VMEM((u_per_w // 2, DPAD), jnp.float32),
            pltpu.SemaphoreType.DMA,
            pltpu.SemaphoreType.DMA,
            pltpu.SemaphoreType.DMA,
        ],
    )
    def k(p_hbm, idx_hbm, utab_hbm, uid_hbm, g_hbm, u_hbm,
          idx_a, idx_b, rows_a, rows_b, uidx_v, urows_v, gsem, ssem_a, ssem_b):
        wid = lax.axis_index("s") * nc + lax.axis_index("c")
        base = wid * per_w
        idx_bufs = (idx_a, idx_b)
        row_bufs = (rows_a, rows_b)
        store_sems = (ssem_a, ssem_b)

        def body(cp, carry):
            for bi in range(2):
                c = cp * 2 + bi
                off = base + c * ch
                idx_v, rows_v, ssem = idx_bufs[bi], row_bufs[bi], store_sems[bi]

                @pl.when(cp >= 1)
                def _():
                    pltpu.make_async_copy(rows_v, g_hbm.at[pl.ds(off, ch)],
                                          ssem).wait()

                pltpu.sync_copy(idx_hbm.at[pl.ds(off, ch)], idx_v)
                pltpu.async_copy(p_hbm.at[idx_v], rows_v, gsem).wait()
                pltpu.async_copy(rows_v, g_hbm.at[pl.ds(off, ch)], ssem)
            return carry

        lax.fori_loop(0, n_ch // 2, body, 0)
        for bi in range(2):
            pltpu.make_async_copy(row_bufs[bi], g_hbm.at[pl.ds(base, ch)],
                                  store_sems[bi]).wait()

        ub = wid * u_per_w
        uh = u_per_w // 2
        pltpu.sync_copy(uid_hbm.at[pl.ds(ub, u_per_w)], uidx_v)
        for half in range(2):
            pltpu.async_copy(utab_hbm.at[uidx_v.at[pl.ds(half * uh, uh)]],
                             urows_v, gsem).wait()
            pltpu.sync_copy(urows_v, u_hbm.at[pl.ds(ub + half * uh, uh)])

    return k(p_tab, idx_flat, user_tab, user_id)


_SG = 8
_BB = 128


def _finish1_body(g_ref, rt_ref, pt_ref, out_ref):
    ptt = pt_ref[...].T
    for si in range(_SG):
        gt = g_ref[:, si, :DM].T
        rv = rt_ref[si:si + 1, :]
        out_ref[si] = (gt + ptt[:, si:si + 1]) * rv


def _finish1(g3, ratings_t, pos_p):
    b, sp, _ = g3.shape
    s = ratings_t.shape[0]
    return pl.pallas_call(
        _finish1_body,
        grid=(sp // _SG, b // _BB),
        in_specs=[
            pl.BlockSpec((_BB, _SG, DPAD), lambda i, j: (j, i, 0)),
            pl.BlockSpec((_SG, _BB), lambda i, j: (i, j)),
            pl.BlockSpec((_SG, DM), lambda i, j: (i, 0)),
        ],
        out_specs=pl.BlockSpec((_SG, DM, _BB), lambda i, j: (i, 0, j)),
        out_shape=jax.ShapeDtypeStruct((s, DM, b), jnp.float32),
    )(g3, ratings_t, pos_p)


def _finish2_body(u_ref, sx_ref, ag_ref, oc_ref, st_ref, at_ref, ot_ref,
                  out2_ref):
    rb = u_ref.shape[0]

    def onehot_emb(x_ref, tab_ref, nv):
        x = x_ref[...]
        i = lax.broadcasted_iota(jnp.int32, (rb, nv), 1).astype(jnp.float32)
        oh = (x == i).astype(jnp.float32)
        return jnp.dot(oh, tab_ref[...], preferred_element_type=jnp.float32)

    e_s = onehot_emb(sx_ref, st_ref, 2)
    e_a = onehot_emb(ag_ref, at_ref, 7)
    e_o = onehot_emb(oc_ref, ot_ref, 21)
    out2_ref[...] = jnp.concatenate([u_ref[...][:, :DM], e_s, e_a, e_o], axis=1)


def _finish2(u, sex_f, age_f, occ_f, sex_table, age_table, occ_table):
    b = u.shape[0]
    rb = 128
    return pl.pallas_call(
        _finish2_body,
        grid=(b // rb,),
        in_specs=[
            pl.BlockSpec((rb, DPAD), lambda i: (i, 0)),
            pl.BlockSpec((rb, 1), lambda i: (i, 0)),
            pl.BlockSpec((rb, 1), lambda i: (i, 0)),
            pl.BlockSpec((rb, 1), lambda i: (i, 0)),
            pl.BlockSpec((2, 1), lambda i: (0, 0)),
            pl.BlockSpec((7, 2), lambda i: (0, 0)),
            pl.BlockSpec((21, 4), lambda i: (0, 0)),
        ],
        out_specs=pl.BlockSpec((rb, DM + 7), lambda i: (i, 0)),
        out_shape=jax.ShapeDtypeStruct((b, DM + 7), jnp.float32),
    )(u, sex_f, age_f, occ_f, sex_table, age_table, occ_table)


def kernel(user_id, sex, age_group, occupation, target_movie_id, sequence_movie_ids,
           sequence_ratings, user_id_table, sex_table, age_group_table, occupation_table,
           movie_table, genre_table, proc_W, proc_b, pos_table):
    b = user_id.shape[0]
    seq = pos_table.shape[0]

    w1 = jnp.pad(proc_W[:DM], ((0, 0), (0, DPAD - DM)))
    w2 = jnp.pad(proc_W[DM:], ((0, 0), (0, DPAD - DM)))
    bpad = jnp.pad(proc_b, (0, DPAD - DM)).reshape(1, DPAD)

    p_tab, u_tab = _precompute(movie_table.T, genre_table.T, w1, w2, bpad,
                               user_id_table.T)

    sp = ((seq + 7) // 8) * 8
    idx_all = jnp.concatenate(
        [sequence_movie_ids, target_movie_id, sequence_movie_ids[:, :sp - seq]],
        axis=1).astype(jnp.int32).reshape(b * sp)

    ratings_ext = jnp.concatenate(
        [sequence_ratings.astype(jnp.float32), jnp.ones((b, 1), jnp.float32)], axis=1)
    pos_ext = jnp.concatenate(
        [pos_table[:seq - 1], jnp.zeros((1, DM), jnp.float32)], axis=0)
    uid = user_id.astype(jnp.int32)
    sex_f = sex.astype(jnp.float32).reshape(b, 1)
    age_f = age_group.astype(jnp.float32).reshape(b, 1)
    occ_f = occupation.astype(jnp.float32).reshape(b, 1)

    ratings_t = ratings_ext.T
    pos_p = jnp.pad(pos_ext, ((0, sp - seq), (0, 0)))

    g2, u = _sc_gather(p_tab, idx_all, u_tab, uid)
    ot = _finish1(g2.reshape(b, sp, DPAD), ratings_t, pos_p)
    out1 = jnp.transpose(ot, (2, 0, 1))
    out2 = _finish2(u, sex_f, age_f, occ_f,
                    sex_table, age_group_table, occupation_table)
    return (out1, out2)


# speedup vs baseline: 1.5027x; 1.1056x over previous
import functools

import jax
import jax.numpy as jnp
from jax import lax
from jax.experimental import pallas as pl
from jax.experimental.pallas import tpu as pltpu
from jax.experimental.pallas import tpu_sc as plsc

DM = 316
DPAD = 384
NG = 18
_DN0 = (((0,), (0,)), ((), ()))


def _proc_body(mt_ref, gt_ref, w1_ref, w2_ref, b_ref, ut_ref, out_ref, uout_ref):
    acc = lax.dot_general(mt_ref[...], w1_ref[...], _DN0,
                          preferred_element_type=jnp.float32)
    acc = acc + lax.dot_general(gt_ref[...], w2_ref[...], _DN0,
                                preferred_element_type=jnp.float32)
    out_ref[...] = jnp.maximum(acc + b_ref[...], 0.0)
    uout_ref[:, :DM] = ut_ref[...].T
    uout_ref[:, DM:] = jnp.zeros_like(uout_ref[:, DM:])


def _precompute(mt_t, gt_t, w1, w2, bpad, ut_t):
    v = mt_t.shape[1]
    rb = 1024
    return pl.pallas_call(
        _proc_body,
        grid=(pl.cdiv(v, rb),),
        in_specs=[
            pl.BlockSpec((DM, rb), lambda i: (0, i)),
            pl.BlockSpec((NG, rb), lambda i: (0, i)),
            pl.BlockSpec((DM, DPAD), lambda i: (0, 0)),
            pl.BlockSpec((NG, DPAD), lambda i: (0, 0)),
            pl.BlockSpec((1, DPAD), lambda i: (0, 0)),
            pl.BlockSpec((DM, rb), lambda i: (0, i)),
        ],
        out_specs=[
            pl.BlockSpec((rb, DPAD), lambda i: (i, 0)),
            pl.BlockSpec((rb, DPAD), lambda i: (i, 0)),
        ],
        out_shape=[
            jax.ShapeDtypeStruct((v, DPAD), jnp.float32),
            jax.ShapeDtypeStruct((v, DPAD), jnp.float32),
        ],
    )(mt_t, gt_t, w1, w2, bpad, ut_t)


def _sc_gather(p_tab, idx_flat, user_tab, user_id):
    nc, ns = 2, 16
    nw = nc * ns
    n = idx_flat.shape[0]
    per_w = n // nw
    ch = 112
    n_ch = per_w // ch
    b = user_id.shape[0]
    u_per_w = b // nw
    mesh = plsc.VectorSubcoreMesh(core_axis_name="c", subcore_axis_name="s",
                                  num_cores=nc, num_subcores=ns)

    @functools.partial(
        pl.kernel,
        mesh=mesh,
        out_type=[
            jax.ShapeDtypeStruct((n, DPAD), jnp.float32),
            jax.ShapeDtypeStruct((b, DPAD), jnp.float32),
        ],
        scratch_types=[
            pltpu.VMEM((ch,), jnp.int32),
            pltpu.VMEM((ch,), jnp.int32),
            pltpu.VMEM((ch, DPAD), jnp.float32),
            pltpu.VMEM((ch, DPAD), jnp.float32),
            pltpu.VMEM((u_per_w,), jnp.int32),
            pltpu.VMEM((u_per_w // 2, DPAD), jnp.float32),
            pltpu.SemaphoreType.DMA,
            pltpu.SemaphoreType.DMA,
            pltpu.SemaphoreType.DMA,
        ],
    )
    def k(p_hbm, idx_hbm, utab_hbm, uid_hbm, g_hbm, u_hbm,
          idx_a, idx_b, rows_a, rows_b, uidx_v, urows_v, gsem, ssem_a, ssem_b):
        wid = lax.axis_index("s") * nc + lax.axis_index("c")
        base = wid * per_w
        idx_bufs = (idx_a, idx_b)
        row_bufs = (rows_a, rows_b)
        store_sems = (ssem_a, ssem_b)

        def body(cp, carry):
            for bi in range(2):
                c = cp * 2 + bi
                off = base + c * ch
                idx_v, rows_v, ssem = idx_bufs[bi], row_bufs[bi], store_sems[bi]

                @pl.when(cp >= 1)
                def _():
                    pltpu.make_async_copy(rows_v, g_hbm.at[pl.ds(off, ch)],
                                          ssem).wait()

                pltpu.sync_copy(idx_hbm.at[pl.ds(off, ch)], idx_v)
                pltpu.async_copy(p_hbm.at[idx_v], rows_v, gsem).wait()
                pltpu.async_copy(rows_v, g_hbm.at[pl.ds(off, ch)], ssem)
            return carry

        lax.fori_loop(0, n_ch // 2, body, 0)
        for bi in range(2):
            pltpu.make_async_copy(row_bufs[bi], g_hbm.at[pl.ds(base, ch)],
                                  store_sems[bi]).wait()

        ub = wid * u_per_w
        uh = u_per_w // 2
        pltpu.sync_copy(uid_hbm.at[pl.ds(ub, u_per_w)], uidx_v)
        for half in range(2):
            pltpu.async_copy(utab_hbm.at[uidx_v.at[pl.ds(half * uh, uh)]],
                             urows_v, gsem).wait()
            pltpu.sync_copy(urows_v, u_hbm.at[pl.ds(ub + half * uh, uh)])

    return k(p_tab, idx_flat, user_tab, user_id)


_SG = 8
_BB = 256


def _finish1_body(g_ref, rt_ref, pt_ref, out_ref):
    ptt = pt_ref[...].T
    for si in range(_SG):
        gt = g_ref[:, si, :DM].T
        rv = rt_ref[si:si + 1, :]
        out_ref[si] = (gt + ptt[:, si:si + 1]) * rv


def _finish1(g3, ratings_t, pos_p):
    b, sp, _ = g3.shape
    s = ratings_t.shape[0]
    return pl.pallas_call(
        _finish1_body,
        grid=(sp // _SG, b // _BB),
        in_specs=[
            pl.BlockSpec((_BB, _SG, DPAD), lambda i, j: (j, i, 0)),
            pl.BlockSpec((_SG, _BB), lambda i, j: (i, j)),
            pl.BlockSpec((_SG, DM), lambda i, j: (i, 0)),
        ],
        out_specs=pl.BlockSpec((_SG, DM, _BB), lambda i, j: (i, 0, j)),
        out_shape=jax.ShapeDtypeStruct((s, DM, b), jnp.float32),
    )(g3, ratings_t, pos_p)


def _finish2_body(u_ref, sx_ref, ag_ref, oc_ref, st_ref, at_ref, ot_ref,
                  out2_ref):
    rb = u_ref.shape[0]

    def onehot_emb(x_ref, tab_ref, nv):
        x = x_ref[...]
        i = lax.broadcasted_iota(jnp.int32, (rb, nv), 1).astype(jnp.float32)
        oh = (x == i).astype(jnp.float32)
        return jnp.dot(oh, tab_ref[...], preferred_element_type=jnp.float32)

    e_s = onehot_emb(sx_ref, st_ref, 2)
    e_a = onehot_emb(ag_ref, at_ref, 7)
    e_o = onehot_emb(oc_ref, ot_ref, 21)
    out2_ref[...] = jnp.concatenate([u_ref[...][:, :DM], e_s, e_a, e_o], axis=1)


def _finish2(u, sex_f, age_f, occ_f, sex_table, age_table, occ_table):
    b = u.shape[0]
    rb = 128
    return pl.pallas_call(
        _finish2_body,
        grid=(b // rb,),
        in_specs=[
            pl.BlockSpec((rb, DPAD), lambda i: (i, 0)),
            pl.BlockSpec((rb, 1), lambda i: (i, 0)),
            pl.BlockSpec((rb, 1), lambda i: (i, 0)),
            pl.BlockSpec((rb, 1), lambda i: (i, 0)),
            pl.BlockSpec((2, 1), lambda i: (0, 0)),
            pl.BlockSpec((7, 2), lambda i: (0, 0)),
            pl.BlockSpec((21, 4), lambda i: (0, 0)),
        ],
        out_specs=pl.BlockSpec((rb, DM + 7), lambda i: (i, 0)),
        out_shape=jax.ShapeDtypeStruct((b, DM + 7), jnp.float32),
    )(u, sex_f, age_f, occ_f, sex_table, age_table, occ_table)


def kernel(user_id, sex, age_group, occupation, target_movie_id, sequence_movie_ids,
           sequence_ratings, user_id_table, sex_table, age_group_table, occupation_table,
           movie_table, genre_table, proc_W, proc_b, pos_table):
    b = user_id.shape[0]
    seq = pos_table.shape[0]

    w1 = jnp.pad(proc_W[:DM], ((0, 0), (0, DPAD - DM)))
    w2 = jnp.pad(proc_W[DM:], ((0, 0), (0, DPAD - DM)))
    bpad = jnp.pad(proc_b, (0, DPAD - DM)).reshape(1, DPAD)

    p_tab, u_tab = _precompute(movie_table.T, genre_table.T, w1, w2, bpad,
                               user_id_table.T)

    sp = ((seq + 7) // 8) * 8
    idx_all = jnp.concatenate(
        [sequence_movie_ids, target_movie_id, sequence_movie_ids[:, :sp - seq]],
        axis=1).astype(jnp.int32).reshape(b * sp)

    ratings_ext = jnp.concatenate(
        [sequence_ratings.astype(jnp.float32), jnp.ones((b, 1), jnp.float32)], axis=1)
    pos_ext = jnp.concatenate(
        [pos_table[:seq - 1], jnp.zeros((1, DM), jnp.float32)], axis=0)
    uid = user_id.astype(jnp.int32)
    sex_f = sex.astype(jnp.float32).reshape(b, 1)
    age_f = age_group.astype(jnp.float32).reshape(b, 1)
    occ_f = occupation.astype(jnp.float32).reshape(b, 1)

    ratings_t = ratings_ext.T
    pos_p = jnp.pad(pos_ext, ((0, sp - seq), (0, 0)))

    g2, u = _sc_gather(p_tab, idx_all, u_tab, uid)
    ot = _finish1(g2.reshape(b, sp, DPAD), ratings_t, pos_p)
    out1 = jnp.transpose(ot, (2, 0, 1))
    out2 = _finish2(u, sex_f, age_f, occ_f,
                    sex_table, age_group_table, occupation_table)
    return (out1, out2)


# speedup vs baseline: 1.5885x; 1.0571x over previous
import functools

import jax
import jax.numpy as jnp
from jax import lax
from jax.experimental import pallas as pl
from jax.experimental.pallas import tpu as pltpu
from jax.experimental.pallas import tpu_sc as plsc

DM = 316
DPAD = 384
NG = 18
_DN0 = (((0,), (0,)), ((), ()))


def _proc_body(mt_ref, gt_ref, w1_ref, w2_ref, b_ref, ut_ref, out_ref, uout_ref):
    acc = lax.dot_general(mt_ref[...], w1_ref[...], _DN0,
                          preferred_element_type=jnp.float32)
    acc = acc + lax.dot_general(gt_ref[...], w2_ref[...], _DN0,
                                preferred_element_type=jnp.float32)
    out_ref[...] = jnp.maximum(acc + b_ref[...], 0.0)
    uout_ref[:, :DM] = ut_ref[...].T
    uout_ref[:, DM:] = jnp.zeros_like(uout_ref[:, DM:])


def _precompute(mt_t, gt_t, w1, w2, bpad, ut_t):
    v = mt_t.shape[1]
    rb = 1024
    return pl.pallas_call(
        _proc_body,
        grid=(pl.cdiv(v, rb),),
        in_specs=[
            pl.BlockSpec((DM, rb), lambda i: (0, i)),
            pl.BlockSpec((NG, rb), lambda i: (0, i)),
            pl.BlockSpec((DM, DPAD), lambda i: (0, 0)),
            pl.BlockSpec((NG, DPAD), lambda i: (0, 0)),
            pl.BlockSpec((1, DPAD), lambda i: (0, 0)),
            pl.BlockSpec((DM, rb), lambda i: (0, i)),
        ],
        out_specs=[
            pl.BlockSpec((rb, DPAD), lambda i: (i, 0)),
            pl.BlockSpec((rb, DPAD), lambda i: (i, 0)),
        ],
        out_shape=[
            jax.ShapeDtypeStruct((v, DPAD), jnp.float32),
            jax.ShapeDtypeStruct((v, DPAD), jnp.float32),
        ],
    )(mt_t, gt_t, w1, w2, bpad, ut_t)


def _sc_gather(p_tab, idx_flat, user_tab, user_id):
    nc, ns = 2, 16
    nw = nc * ns
    n = idx_flat.shape[0]
    per_w = n // nw
    ch = 112
    n_ch = per_w // ch
    b = user_id.shape[0]
    u_per_w = b // nw
    mesh = plsc.VectorSubcoreMesh(core_axis_name="c", subcore_axis_name="s",
                                  num_cores=nc, num_subcores=ns)

    @functools.partial(
        pl.kernel,
        mesh=mesh,
        out_type=[
            jax.ShapeDtypeStruct((n, DPAD), jnp.float32),
            jax.ShapeDtypeStruct((b, DPAD), jnp.float32),
        ],
        scratch_types=[
            pltpu.VMEM((ch,), jnp.int32),
            pltpu.VMEM((ch,), jnp.int32),
            pltpu.VMEM((ch, DPAD), jnp.float32),
            pltpu.VMEM((ch, DPAD), jnp.float32),
            pltpu.VMEM((u_per_w,), jnp.int32),
            pltpu.VMEM((u_per_w // 2, DPAD), jnp.float32),
            pltpu.SemaphoreType.DMA,
            pltpu.SemaphoreType.DMA,
            pltpu.SemaphoreType.DMA,
        ],
    )
    def k(p_hbm, idx_hbm, utab_hbm, uid_hbm, g_hbm, u_hbm,
          idx_a, idx_b, rows_a, rows_b, uidx_v, urows_v, gsem, ssem_a, ssem_b):
        wid = lax.axis_index("s") * nc + lax.axis_index("c")
        base = wid * per_w
        idx_bufs = (idx_a, idx_b)
        row_bufs = (rows_a, rows_b)
        store_sems = (ssem_a, ssem_b)

        def body(cp, carry):
            for bi in range(2):
                c = cp * 2 + bi
                off = base + c * ch
                idx_v, rows_v, ssem = idx_bufs[bi], row_bufs[bi], store_sems[bi]

                @pl.when(cp >= 1)
                def _():
                    pltpu.make_async_copy(rows_v, g_hbm.at[pl.ds(off, ch)],
                                          ssem).wait()

                pltpu.sync_copy(idx_hbm.at[pl.ds(off, ch)], idx_v)
                pltpu.async_copy(p_hbm.at[idx_v], rows_v, gsem).wait()
                pltpu.async_copy(rows_v, g_hbm.at[pl.ds(off, ch)], ssem)
            return carry

        lax.fori_loop(0, n_ch // 2, body, 0)
        for bi in range(2):
            pltpu.make_async_copy(row_bufs[bi], g_hbm.at[pl.ds(base, ch)],
                                  store_sems[bi]).wait()

        ub = wid * u_per_w
        uh = u_per_w // 2
        pltpu.sync_copy(uid_hbm.at[pl.ds(ub, u_per_w)], uidx_v)
        for half in range(2):
            pltpu.async_copy(utab_hbm.at[uidx_v.at[pl.ds(half * uh, uh)]],
                             urows_v, gsem).wait()
            pltpu.sync_copy(urows_v, u_hbm.at[pl.ds(ub + half * uh, uh)])

    return k(p_tab, idx_flat, user_tab, user_id)


_SG = 8
_BB = 512


def _finish1_body(g_ref, rt_ref, pt_ref, out_ref):
    ptt = pt_ref[...].T
    for si in range(_SG):
        gt = g_ref[:, si, :DM].T
        rv = rt_ref[si:si + 1, :]
        out_ref[si] = (gt + ptt[:, si:si + 1]) * rv


def _finish1(g3, ratings_t, pos_p):
    b, sp, _ = g3.shape
    s = ratings_t.shape[0]
    return pl.pallas_call(
        _finish1_body,
        grid=(sp // _SG, b // _BB),
        in_specs=[
            pl.BlockSpec((_BB, _SG, DPAD), lambda i, j: (j, i, 0)),
            pl.BlockSpec((_SG, _BB), lambda i, j: (i, j)),
            pl.BlockSpec((_SG, DM), lambda i, j: (i, 0)),
        ],
        out_specs=pl.BlockSpec((_SG, DM, _BB), lambda i, j: (i, 0, j)),
        out_shape=jax.ShapeDtypeStruct((s, DM, b), jnp.float32),
    )(g3, ratings_t, pos_p)


def _finish2_body(u_ref, sx_ref, ag_ref, oc_ref, st_ref, at_ref, ot_ref,
                  out2_ref):
    rb = u_ref.shape[0]

    def onehot_emb(x_ref, tab_ref, nv):
        x = x_ref[...]
        i = lax.broadcasted_iota(jnp.int32, (rb, nv), 1).astype(jnp.float32)
        oh = (x == i).astype(jnp.float32)
        return jnp.dot(oh, tab_ref[...], preferred_element_type=jnp.float32)

    e_s = onehot_emb(sx_ref, st_ref, 2)
    e_a = onehot_emb(ag_ref, at_ref, 7)
    e_o = onehot_emb(oc_ref, ot_ref, 21)
    out2_ref[...] = jnp.concatenate([u_ref[...][:, :DM], e_s, e_a, e_o], axis=1)


def _finish2(u, sex_f, age_f, occ_f, sex_table, age_table, occ_table):
    b = u.shape[0]
    rb = 128
    return pl.pallas_call(
        _finish2_body,
        grid=(b // rb,),
        in_specs=[
            pl.BlockSpec((rb, DPAD), lambda i: (i, 0)),
            pl.BlockSpec((rb, 1), lambda i: (i, 0)),
            pl.BlockSpec((rb, 1), lambda i: (i, 0)),
            pl.BlockSpec((rb, 1), lambda i: (i, 0)),
            pl.BlockSpec((2, 1), lambda i: (0, 0)),
            pl.BlockSpec((7, 2), lambda i: (0, 0)),
            pl.BlockSpec((21, 4), lambda i: (0, 0)),
        ],
        out_specs=pl.BlockSpec((rb, DM + 7), lambda i: (i, 0)),
        out_shape=jax.ShapeDtypeStruct((b, DM + 7), jnp.float32),
    )(u, sex_f, age_f, occ_f, sex_table, age_table, occ_table)


def kernel(user_id, sex, age_group, occupation, target_movie_id, sequence_movie_ids,
           sequence_ratings, user_id_table, sex_table, age_group_table, occupation_table,
           movie_table, genre_table, proc_W, proc_b, pos_table):
    b = user_id.shape[0]
    seq = pos_table.shape[0]

    w1 = jnp.pad(proc_W[:DM], ((0, 0), (0, DPAD - DM)))
    w2 = jnp.pad(proc_W[DM:], ((0, 0), (0, DPAD - DM)))
    bpad = jnp.pad(proc_b, (0, DPAD - DM)).reshape(1, DPAD)

    p_tab, u_tab = _precompute(movie_table.T, genre_table.T, w1, w2, bpad,
                               user_id_table.T)

    sp = ((seq + 7) // 8) * 8
    idx_all = jnp.concatenate(
        [sequence_movie_ids, target_movie_id, sequence_movie_ids[:, :sp - seq]],
        axis=1).astype(jnp.int32).reshape(b * sp)

    ratings_ext = jnp.concatenate(
        [sequence_ratings.astype(jnp.float32), jnp.ones((b, 1), jnp.float32)], axis=1)
    pos_ext = jnp.concatenate(
        [pos_table[:seq - 1], jnp.zeros((1, DM), jnp.float32)], axis=0)
    uid = user_id.astype(jnp.int32)
    sex_f = sex.astype(jnp.float32).reshape(b, 1)
    age_f = age_group.astype(jnp.float32).reshape(b, 1)
    occ_f = occupation.astype(jnp.float32).reshape(b, 1)

    ratings_t = ratings_ext.T
    pos_p = jnp.pad(pos_ext, ((0, sp - seq), (0, 0)))

    g2, u = _sc_gather(p_tab, idx_all, u_tab, uid)
    ot = _finish1(g2.reshape(b, sp, DPAD), ratings_t, pos_p)
    out1 = jnp.transpose(ot, (2, 0, 1))
    out2 = _finish2(u, sex_f, age_f, occ_f,
                    sex_table, age_group_table, occupation_table)
    return (out1, out2)


# speedup vs baseline: 1.6043x; 1.0099x over previous
import functools

import jax
import jax.numpy as jnp
from jax import lax
from jax.experimental import pallas as pl
from jax.experimental.pallas import tpu as pltpu
from jax.experimental.pallas import tpu_sc as plsc

DM = 316
DPAD = 384
NG = 18
_DN0 = (((0,), (0,)), ((), ()))


def _proc_body(mt_ref, gt_ref, w1_ref, w2_ref, b_ref, ut_ref, out_ref, uout_ref):
    acc = lax.dot_general(mt_ref[...], w1_ref[...], _DN0,
                          preferred_element_type=jnp.float32)
    acc = acc + lax.dot_general(gt_ref[...], w2_ref[...], _DN0,
                                preferred_element_type=jnp.float32)
    out_ref[...] = jnp.maximum(acc + b_ref[...], 0.0)
    uout_ref[:, :DM] = ut_ref[...].T
    uout_ref[:, DM:] = jnp.zeros_like(uout_ref[:, DM:])


def _precompute(mt_t, gt_t, w1, w2, bpad, ut_t):
    v = mt_t.shape[1]
    rb = 1024
    return pl.pallas_call(
        _proc_body,
        grid=(pl.cdiv(v, rb),),
        in_specs=[
            pl.BlockSpec((DM, rb), lambda i: (0, i)),
            pl.BlockSpec((NG, rb), lambda i: (0, i)),
            pl.BlockSpec((DM, DPAD), lambda i: (0, 0)),
            pl.BlockSpec((NG, DPAD), lambda i: (0, 0)),
            pl.BlockSpec((1, DPAD), lambda i: (0, 0)),
            pl.BlockSpec((DM, rb), lambda i: (0, i)),
        ],
        out_specs=[
            pl.BlockSpec((rb, DPAD), lambda i: (i, 0)),
            pl.BlockSpec((rb, DPAD), lambda i: (i, 0)),
        ],
        out_shape=[
            jax.ShapeDtypeStruct((v, DPAD), jnp.float32),
            jax.ShapeDtypeStruct((v, DPAD), jnp.float32),
        ],
    )(mt_t, gt_t, w1, w2, bpad, ut_t)


def _sc_gather(p_tab, idx_flat, user_tab, user_id):
    nc, ns = 2, 16
    nw = nc * ns
    n = idx_flat.shape[0]
    per_w = n // nw
    ch = 112
    n_ch = per_w // ch
    b = user_id.shape[0]
    u_per_w = b // nw
    mesh = plsc.VectorSubcoreMesh(core_axis_name="c", subcore_axis_name="s",
                                  num_cores=nc, num_subcores=ns)

    @functools.partial(
        pl.kernel,
        mesh=mesh,
        out_type=[
            jax.ShapeDtypeStruct((n, DPAD), jnp.float32),
            jax.ShapeDtypeStruct((b, DPAD), jnp.float32),
        ],
        scratch_types=[
            pltpu.VMEM((ch,), jnp.int32),
            pltpu.VMEM((ch,), jnp.int32),
            pltpu.VMEM((ch, DPAD), jnp.float32),
            pltpu.VMEM((ch, DPAD), jnp.float32),
            pltpu.VMEM((u_per_w,), jnp.int32),
            pltpu.VMEM((u_per_w // 2, DPAD), jnp.float32),
            pltpu.SemaphoreType.DMA,
            pltpu.SemaphoreType.DMA,
            pltpu.SemaphoreType.DMA,
        ],
    )
    def k(p_hbm, idx_hbm, utab_hbm, uid_hbm, g_hbm, u_hbm,
          idx_a, idx_b, rows_a, rows_b, uidx_v, urows_v, gsem, ssem_a, ssem_b):
        wid = lax.axis_index("s") * nc + lax.axis_index("c")
        base = wid * per_w
        idx_bufs = (idx_a, idx_b)
        row_bufs = (rows_a, rows_b)
        store_sems = (ssem_a, ssem_b)

        def body(cp, carry):
            for bi in range(2):
                c = cp * 2 + bi
                off = base + c * ch
                idx_v, rows_v, ssem = idx_bufs[bi], row_bufs[bi], store_sems[bi]

                @pl.when(cp >= 1)
                def _():
                    pltpu.make_async_copy(rows_v, g_hbm.at[pl.ds(off, ch)],
                                          ssem).wait()

                pltpu.sync_copy(idx_hbm.at[pl.ds(off, ch)], idx_v)
                pltpu.async_copy(p_hbm.at[idx_v], rows_v, gsem).wait()
                pltpu.async_copy(rows_v, g_hbm.at[pl.ds(off, ch)], ssem)
            return carry

        lax.fori_loop(0, n_ch // 2, body, 0)
        for bi in range(2):
            pltpu.make_async_copy(row_bufs[bi], g_hbm.at[pl.ds(base, ch)],
                                  store_sems[bi]).wait()

        ub = wid * u_per_w
        uh = u_per_w // 2
        pltpu.sync_copy(uid_hbm.at[pl.ds(ub, u_per_w)], uidx_v)
        for half in range(2):
            pltpu.async_copy(utab_hbm.at[uidx_v.at[pl.ds(half * uh, uh)]],
                             urows_v, gsem).wait()
            pltpu.sync_copy(urows_v, u_hbm.at[pl.ds(ub + half * uh, uh)])

    return k(p_tab, idx_flat, user_tab, user_id)


_SG = 8
_BB = 1024


def _finish1_body(g_ref, rt_ref, pt_ref, out_ref):
    ptt = pt_ref[...].T
    for si in range(_SG):
        gt = g_ref[:, si, :DM].T
        rv = rt_ref[si:si + 1, :]
        out_ref[si] = (gt + ptt[:, si:si + 1]) * rv


def _finish1(g3, ratings_t, pos_p):
    b, sp, _ = g3.shape
    s = ratings_t.shape[0]
    return pl.pallas_call(
        _finish1_body,
        grid=(sp // _SG, b // _BB),
        in_specs=[
            pl.BlockSpec((_BB, _SG, DPAD), lambda i, j: (j, i, 0)),
            pl.BlockSpec((_SG, _BB), lambda i, j: (i, j)),
            pl.BlockSpec((_SG, DM), lambda i, j: (i, 0)),
        ],
        out_specs=pl.BlockSpec((_SG, DM, _BB), lambda i, j: (i, 0, j)),
        out_shape=jax.ShapeDtypeStruct((s, DM, b), jnp.float32),
    )(g3, ratings_t, pos_p)


def _finish2_body(u_ref, sx_ref, ag_ref, oc_ref, st_ref, at_ref, ot_ref,
                  out2_ref):
    rb = u_ref.shape[0]

    def onehot_emb(x_ref, tab_ref, nv):
        x = x_ref[...]
        i = lax.broadcasted_iota(jnp.int32, (rb, nv), 1).astype(jnp.float32)
        oh = (x == i).astype(jnp.float32)
        return jnp.dot(oh, tab_ref[...], preferred_element_type=jnp.float32)

    e_s = onehot_emb(sx_ref, st_ref, 2)
    e_a = onehot_emb(ag_ref, at_ref, 7)
    e_o = onehot_emb(oc_ref, ot_ref, 21)
    out2_ref[...] = jnp.concatenate([u_ref[...][:, :DM], e_s, e_a, e_o], axis=1)


def _finish2(u, sex_f, age_f, occ_f, sex_table, age_table, occ_table):
    b = u.shape[0]
    rb = 128
    return pl.pallas_call(
        _finish2_body,
        grid=(b // rb,),
        in_specs=[
            pl.BlockSpec((rb, DPAD), lambda i: (i, 0)),
            pl.BlockSpec((rb, 1), lambda i: (i, 0)),
            pl.BlockSpec((rb, 1), lambda i: (i, 0)),
            pl.BlockSpec((rb, 1), lambda i: (i, 0)),
            pl.BlockSpec((2, 1), lambda i: (0, 0)),
            pl.BlockSpec((7, 2), lambda i: (0, 0)),
            pl.BlockSpec((21, 4), lambda i: (0, 0)),
        ],
        out_specs=pl.BlockSpec((rb, DM + 7), lambda i: (i, 0)),
        out_shape=jax.ShapeDtypeStruct((b, DM + 7), jnp.float32),
    )(u, sex_f, age_f, occ_f, sex_table, age_table, occ_table)


def kernel(user_id, sex, age_group, occupation, target_movie_id, sequence_movie_ids,
           sequence_ratings, user_id_table, sex_table, age_group_table, occupation_table,
           movie_table, genre_table, proc_W, proc_b, pos_table):
    b = user_id.shape[0]
    seq = pos_table.shape[0]

    w1 = jnp.pad(proc_W[:DM], ((0, 0), (0, DPAD - DM)))
    w2 = jnp.pad(proc_W[DM:], ((0, 0), (0, DPAD - DM)))
    bpad = jnp.pad(proc_b, (0, DPAD - DM)).reshape(1, DPAD)

    p_tab, u_tab = _precompute(movie_table.T, genre_table.T, w1, w2, bpad,
                               user_id_table.T)

    sp = ((seq + 7) // 8) * 8
    idx_all = jnp.concatenate(
        [sequence_movie_ids, target_movie_id, sequence_movie_ids[:, :sp - seq]],
        axis=1).astype(jnp.int32).reshape(b * sp)

    ratings_ext = jnp.concatenate(
        [sequence_ratings.astype(jnp.float32), jnp.ones((b, 1), jnp.float32)], axis=1)
    pos_ext = jnp.concatenate(
        [pos_table[:seq - 1], jnp.zeros((1, DM), jnp.float32)], axis=0)
    uid = user_id.astype(jnp.int32)
    sex_f = sex.astype(jnp.float32).reshape(b, 1)
    age_f = age_group.astype(jnp.float32).reshape(b, 1)
    occ_f = occupation.astype(jnp.float32).reshape(b, 1)

    ratings_t = ratings_ext.T
    pos_p = jnp.pad(pos_ext, ((0, sp - seq), (0, 0)))

    g2, u = _sc_gather(p_tab, idx_all, u_tab, uid)
    ot = _finish1(g2.reshape(b, sp, DPAD), ratings_t, pos_p)
    out1 = jnp.transpose(ot, (2, 0, 1))
    out2 = _finish2(u, sex_f, age_f, occ_f,
                    sex_table, age_group_table, occupation_table)
    return (out1, out2)
